# baseline (speedup 1.0000x reference)
import functools

import jax
import jax.numpy as jnp
from jax import lax
from jax.experimental import pallas as pl
from jax.experimental.pallas import tpu as pltpu
from jax.experimental.pallas import tpu_sc as plsc

VOCAB = 1000000
N_WORD = 64
B = 4096
L = 50

_NC = 2
_NS = 16
_NW = _NC * _NS

_TOTAL = B * L
_PER_W = _TOTAL // _NW
_BPW = B // _NW
_CHUNK = 400
_NSTEP = _PER_W // _CHUNK
_NBUF = 4

_MESH = plsc.VectorSubcoreMesh(core_axis_name="c", subcore_axis_name="s")


@functools.partial(
    pl.kernel,
    mesh=_MESH,
    out_type=jax.ShapeDtypeStruct((_TOTAL,), jnp.int32),
    scratch_types=[
        pltpu.VMEM((L, _BPW), jnp.int32),
        pltpu.VMEM((_PER_W,), jnp.int32),
    ],
    compiler_params=pltpu.CompilerParams(needs_layout_passes=False),
)
def _flatten(idx_hbm, out_hbm, idx_blk, idx_v):
  wid = lax.axis_index("s") * _NC + lax.axis_index("c")
  b0 = wid * _BPW

  pltpu.sync_copy(idx_hbm.at[:, pl.ds(b0, _BPW)], idx_blk)

  def reorder(g, _):
    p = g * 16 + lax.iota(jnp.int32, 16)
    bb = lax.shift_right_logical(p * 83887, 22)
    t = p - bb * L
    idx_v[pl.ds(g * 16, 16)] = plsc.load_gather(idx_blk, [t, bb])
    return _
  lax.fori_loop(0, _PER_W // 16, reorder, 0)

  pltpu.sync_copy(idx_v, out_hbm.at[pl.ds(wid * _PER_W, _PER_W)])


@functools.partial(
    pl.kernel,
    mesh=_MESH,
    out_type=jax.ShapeDtypeStruct((_TOTAL, N_WORD), jnp.float32),
    scratch_types=[
        pltpu.VMEM((_PER_W,), jnp.int32),
        [pltpu.VMEM((_CHUNK, N_WORD), jnp.float32) for _ in range(_NBUF)],
        [pltpu.SemaphoreType.DMA for _ in range(_NBUF)],
        [pltpu.SemaphoreType.DMA for _ in range(_NBUF)],
    ],
    compiler_params=pltpu.CompilerParams(use_tc_tiling_on_sc=False),
)
def _gather(idx_hbm, table_hbm, out_hbm, idx_v, rows, gsem, ssem):
  wid = lax.axis_index("s") * _NC + lax.axis_index("c")
  base = wid * _PER_W

  pltpu.sync_copy(idx_hbm.at[pl.ds(base, _PER_W)], idx_v)

  def issue_gather(step, buf):
    return pltpu.async_copy(
        table_hbm.at[idx_v.at[pl.ds(step * _CHUNK, _CHUNK)]],
        rows[buf], gsem[buf])

  def issue_store(step, buf):
    return pltpu.async_copy(
        rows[buf], out_hbm.at[pl.ds(base + step * _CHUNK, _CHUNK)],
        ssem[buf])

  gh = [None] * _NBUF
  sh = [None] * _NBUF
  for b in range(_NBUF):
    gh[b] = issue_gather(b, b)

  for i in range(_NSTEP):
    b = i % _NBUF
    gh[b].wait()
    sh[b] = issue_store(i, b)
    j = i - 1 + _NBUF
    if i >= 1 and j < _NSTEP:
      pb = (i - 1) % _NBUF
      sh[pb].wait()
      gh[pb] = issue_gather(j, pb)

  for i in range(_NSTEP - _NBUF, _NSTEP):
    sh[i % _NBUF].wait()


@jax.jit
def kernel(val_tok, embedding_weight):
  idx = _flatten(val_tok.T.astype(jnp.int32))
  out = _gather(idx, embedding_weight)
  return out.reshape(B, L, N_WORD)


# speedup vs baseline: 1.3245x; 1.3245x over previous
import functools

import jax
import jax.numpy as jnp
from jax import lax
from jax.experimental import pallas as pl
from jax.experimental.pallas import tpu as pltpu
from jax.experimental.pallas import tpu_sc as plsc

VOCAB = 1000000
N_WORD = 64
B = 4096
L = 50

_NC = 2
_NS = 16
_NW = _NC * _NS

_TOTAL = B * L
_PER_W = _TOTAL // _NW
_BCHUNK = 16
_CHUNK = _BCHUNK * L
_NSTEP = _PER_W // _CHUNK


def _make_gather():
  mesh = plsc.VectorSubcoreMesh(core_axis_name="c", subcore_axis_name="s")

  @functools.partial(
      pl.kernel,
      mesh=mesh,
      out_type=jax.ShapeDtypeStruct((_TOTAL, N_WORD), jnp.float32),
      scratch_types=[
          pltpu.VMEM((_CHUNK,), jnp.int32),
          pltpu.VMEM((_CHUNK, N_WORD), jnp.float32),
          pltpu.SemaphoreType.DMA,
      ],
  )
  def emb_gather(idx_hbm, table_hbm, out_hbm, idx_v, rows_v, sem):
    wid = lax.axis_index("s") * _NC + lax.axis_index("c")
    row_base = wid * _PER_W

    for j in range(_NSTEP):
      off = row_base + j * _CHUNK
      pltpu.sync_copy(idx_hbm.at[pl.ds(off, _CHUNK)], idx_v)

      def issue_group(g, _):
        v = idx_v[pl.ds(g * 16, 16)]
        for k in range(16):
          pltpu.async_copy(
              table_hbm.at[pl.ds(v[k], 1)],
              rows_v.at[pl.ds(g * 16 + k, 1)],
              sem,
          )
        return _
      lax.fori_loop(0, _CHUNK // 16, issue_group, 0)

      pltpu.make_async_copy(
          out_hbm.at[pl.ds(off, _CHUNK)], rows_v, sem).wait()
      pltpu.sync_copy(rows_v, out_hbm.at[pl.ds(off, _CHUNK)])

  return emb_gather


_gather = _make_gather()


@jax.jit
def kernel(val_tok, embedding_weight):
  idx = val_tok.reshape(-1).astype(jnp.int32)
  out = _gather(idx, embedding_weight)
  return out.reshape(B, L, N_WORD)
